# baseline (speedup 1.0000x reference)
import functools

import jax
import jax.numpy as jnp
from jax import lax
from jax.experimental import pallas as pl
from jax.experimental.pallas import tpu as pltpu
from jax.experimental.pallas import tpu_sc as plsc

_N = 10000
_E = 320000
_H = 128
_NG = 64
_NC = 2
_NS = 16
_NW = _NC * _NS
_CK = 40
_NCH = _E // _CK
_CPW = _NCH // _NW
_NPW = 624
_NTAIL = _N - _NS * _NPW


def _node_embed_body(x_ref, w_ref, b_ref, o_ref):
    o_ref[...] = jnp.maximum(
        jnp.dot(x_ref[...], w_ref[...], preferred_element_type=jnp.float32)
        + b_ref[...], 0.0)


def _node_embed(x, w_t, b):
    return pl.pallas_call(
        _node_embed_body,
        grid=(10,),
        in_specs=[
            pl.BlockSpec((1000, _H), lambda i: (i, 0)),
            pl.BlockSpec((_H, _H), lambda i: (0, 0)),
            pl.BlockSpec((1, _H), lambda i: (0, 0)),
        ],
        out_specs=pl.BlockSpec((1000, _H), lambda i: (i, 0)),
        out_shape=jax.ShapeDtypeStruct((_N, _H), jnp.float32),
    )(x, w_t, b)


def _edge_body(ea_ref, w1_ref, b1_ref, w2_ref, b2_ref, o_ref):
    t = jnp.maximum(
        jnp.dot(ea_ref[...], w1_ref[...], preferred_element_type=jnp.float32)
        + b1_ref[...], 0.0)
    o_ref[...] = (
        jnp.dot(t, w2_ref[...], preferred_element_type=jnp.float32)
        + b2_ref[...])


def _edge_transform(ea, w1_t, b1, w2_t, b2):
    blk = 4000
    return pl.pallas_call(
        _edge_body,
        grid=(_E // blk,),
        in_specs=[
            pl.BlockSpec((blk, 16), lambda i: (i, 0)),
            pl.BlockSpec((16, _H), lambda i: (0, 0)),
            pl.BlockSpec((1, _H), lambda i: (0, 0)),
            pl.BlockSpec((_H, _H), lambda i: (0, 0)),
            pl.BlockSpec((1, _H), lambda i: (0, 0)),
        ],
        out_specs=pl.BlockSpec((blk, _H), lambda i: (i, 0)),
        out_shape=jax.ShapeDtypeStruct((_E, _H), jnp.float32),
    )(ea, w1_t, b1, w2_t, b2)


def _update_body(h_ref, a_ref, ev_ref, w1_ref, b1_ref, w2_ref, b2_ref, o_ref):
    u = h_ref[...] * ev_ref[...] + a_ref[0] + a_ref[1]
    t = jnp.maximum(
        jnp.dot(u, w1_ref[...], preferred_element_type=jnp.float32)
        + b1_ref[...], 0.0)
    o_ref[...] = jnp.maximum(
        jnp.dot(t, w2_ref[...], preferred_element_type=jnp.float32)
        + b2_ref[...], 0.0)


def _node_update(h, aggr2, epsv, w1_t, b1, w2_t, b2):
    return pl.pallas_call(
        _update_body,
        grid=(10,),
        in_specs=[
            pl.BlockSpec((1000, _H), lambda i: (i, 0)),
            pl.BlockSpec((_NC, 1000, _H), lambda i: (0, i, 0)),
            pl.BlockSpec((1, _H), lambda i: (0, 0)),
            pl.BlockSpec((_H, _H), lambda i: (0, 0)),
            pl.BlockSpec((1, _H), lambda i: (0, 0)),
            pl.BlockSpec((_H, _H), lambda i: (0, 0)),
            pl.BlockSpec((1, _H), lambda i: (0, 0)),
        ],
        out_specs=pl.BlockSpec((1000, _H), lambda i: (i, 0)),
        out_shape=jax.ShapeDtypeStruct((_N, _H), jnp.float32),
    )(h, aggr2, epsv, w1_t, b1, w2_t, b2)


def _head_body(ge_ref, gf_ref, gw1_ref, gb1_ref, gw2_ref, gb2_ref,
               w1a_ref, w1b_ref, b1_ref, w2_ref, b2_ref, w3_ref, b3_ref,
               o_ref):
    g = jnp.maximum(
        jnp.dot(gf_ref[...], gw1_ref[...], preferred_element_type=jnp.float32)
        + gb1_ref[...], 0.0)
    g = jnp.maximum(
        jnp.dot(g, gw2_ref[...], preferred_element_type=jnp.float32)
        + gb2_ref[...], 0.0)
    z = (jnp.dot(ge_ref[...], w1a_ref[...], preferred_element_type=jnp.float32)
         + jnp.dot(g, w1b_ref[...], preferred_element_type=jnp.float32)
         + b1_ref[...])
    z = jnp.maximum(z, 0.0)
    z = jnp.maximum(
        jnp.dot(z, w2_ref[...], preferred_element_type=jnp.float32)
        + b2_ref[...], 0.0)
    o_ref[...] = (
        jnp.dot(z, w3_ref[...], preferred_element_type=jnp.float32)
        + b3_ref[...])


def _head(ge, gf, gw1_t, gb1, gw2_t, gb2, w1a, w1b, b1, w2_t, b2, w3_t, b3):
    return pl.pallas_call(
        _head_body,
        out_shape=jax.ShapeDtypeStruct((_NG, 1), jnp.float32),
    )(ge, gf, gw1_t, gb1, gw2_t, gb2, w1a, w1b, b1, w2_t, b2, w3_t, b3)


def _msgpass_body(h_hbm, e_hbm, src1_hbm, dst1_hbm, out_hbm,
                  sg, sg2, dg, dg2, rows_v, msg_v, aggr_sh, sem_g, sem_e):
    cid = lax.axis_index("c")
    sid = lax.axis_index("s")
    w = cid * _NS + sid

    zero16 = jnp.zeros((16,), jnp.float32)

    def zrow(i, _):
        for c in range(8):
            msg_v[0, i, pl.ds(c * 16, 16)] = zero16
        return 0
    lax.fori_loop(0, _CK, zrow, 0)
    base_n = sid * _NPW
    off_n = 0
    while off_n < _NPW:
        n = min(_CK, _NPW - off_n)
        pltpu.sync_copy(msg_v.at[0, pl.ds(0, n)],
                        aggr_sh.at[pl.ds(base_n + off_n, n)])
        off_n += n

    @pl.when(sid == 0)
    def _():
        pltpu.sync_copy(msg_v.at[0, pl.ds(0, _NTAIL)],
                        aggr_sh.at[pl.ds(_NS * _NPW, _NTAIL)])

    cb = _CPW * w
    sgs = (sg, sg2)
    dgs = (dg, dg2)

    def load_idx(j, k):
        base = (cb + j) * _CK
        pltpu.sync_copy(src1_hbm.at[pl.ds(base, _CK)], sgs[k])
        pltpu.sync_copy(dst1_hbm.at[pl.ds(base, _CK)], dgs[k])

    def issue_in(j, k):
        pltpu.async_copy(h_hbm.at[sgs[k]], rows_v.at[k], sem_g)
        pltpu.async_copy(e_hbm.at[pl.ds((cb + j) * _CK, _CK)],
                         msg_v.at[k], sem_e)

    def drain_in(j, k):
        pltpu.make_async_copy(h_hbm.at[sgs[k]], rows_v.at[k], sem_g).wait()
        pltpu.make_async_copy(e_hbm.at[pl.ds((cb + j) * _CK, _CK)],
                              msg_v.at[k], sem_e).wait()

    def compute_scatter(k):
        def row(i, _):
            for c in range(8):
                s_ = pl.ds(c * 16, 16)
                msg_v[k, i, s_] = jnp.maximum(
                    rows_v[k, i, s_] + msg_v[k, i, s_], 0.0)
            return 0
        lax.fori_loop(0, _CK, row, 0)
        pltpu.sync_copy(msg_v.at[k], aggr_sh.at[dgs[k]], add=True)

    load_idx(0, 0)
    issue_in(0, 0)
    plsc.subcore_barrier()

    def pair(jj, _):
        j = 2 * jj
        load_idx(j + 1, 1)
        issue_in(j + 1, 1)
        drain_in(j, 0)
        compute_scatter(0)

        @pl.when(j + 2 < _CPW)
        def _():
            load_idx(j + 2, 0)
            issue_in(j + 2, 0)
        drain_in(j + 1, 1)
        compute_scatter(1)
        return 0
    lax.fori_loop(0, _CPW // 2, pair, 0)

    plsc.subcore_barrier()
    pltpu.sync_copy(aggr_sh.at[pl.ds(base_n, _NPW)],
                    out_hbm.at[cid, pl.ds(base_n, _NPW)])

    @pl.when(sid == 0)
    def _():
        pltpu.sync_copy(aggr_sh.at[pl.ds(_NS * _NPW, _NTAIL)],
                        out_hbm.at[cid, pl.ds(_NS * _NPW, _NTAIL)])


@functools.cache
def _build_msgpass():
    return pl.kernel(
        _msgpass_body,
        out_type=jax.ShapeDtypeStruct((_NC, _N, _H), jnp.float32),
        mesh=plsc.VectorSubcoreMesh(core_axis_name="c", subcore_axis_name="s",
                                    num_cores=_NC, num_subcores=_NS),
        scratch_types=[
            pltpu.VMEM((_CK,), jnp.int32),
            pltpu.VMEM((_CK,), jnp.int32),
            pltpu.VMEM((_CK,), jnp.int32),
            pltpu.VMEM((_CK,), jnp.int32),
            pltpu.VMEM((2, _CK, _H), jnp.float32),
            pltpu.VMEM((2, _CK, _H), jnp.float32),
            pltpu.VMEM_SHARED((_N, _H), jnp.float32),
            pltpu.SemaphoreType.DMA,
            pltpu.SemaphoreType.DMA,
        ],
    )


def _msgpass(h, e_l, src1, dst1):
    return _build_msgpass()(h, e_l, src1, dst1)


_TILE_G = 40
_TILE_R = _TILE_G * 8


def _pool_core0(h_hbm, b_hbm, t_hbm, out_hbm,
                hbuf, bbuf, gmax_v, s1_v, s2_v, tmp_a, tmp_b,
                tbuf, sh_a, sh_b, gmax_glob):
    sid = lax.axis_index("s")
    gcnt = 78 + jnp.where(sid < 2, 1, 0)
    gbase = 78 * sid + jnp.minimum(sid, 2)
    rbase = gbase * 8

    pltpu.sync_copy(t_hbm, tbuf)
    tv = tbuf[...]

    zero16 = jnp.zeros((16,), jnp.float32)
    neg16 = jnp.full((16,), -3e38, jnp.float32)

    def zi(i, _):
        for c in range(8):
            s_ = pl.ds(c * 16, 16)
            gmax_v[i, s_] = neg16
            s1_v[i, s_] = zero16
            s2_v[i, s_] = zero16
        return 0
    lax.fori_loop(0, _NG, zi, 0)

    def stage(tile):
        rs = rbase + _TILE_R * tile
        pltpu.sync_copy(h_hbm.at[pl.ds(rs, _TILE_R)], hbuf)
        pltpu.sync_copy(b_hbm.at[pl.ds(rs, _TILE_R)],
                        bbuf.at[pl.ds(0, _TILE_R)])
        return jnp.minimum(_TILE_G, gcnt - _TILE_G * tile)

    def row_a(jb, _):
        bvec = bbuf[pl.ds(8 * jb, 16)]
        for lane in range(8):
            g = bvec[lane]
            i = 8 * jb + lane
            for c in range(8):
                s_ = pl.ds(c * 16, 16)
                gmax_v[g, s_] = jnp.maximum(gmax_v[g, s_], hbuf[i, s_] * tv)
        return 0
    for tile in range(2):
        ngt = stage(tile)
        lax.fori_loop(0, ngt, row_a, 0)

    pltpu.sync_copy(gmax_v, sh_a.at[sid])
    plsc.subcore_barrier()

    gslice = pl.ds(8 * sid, 8)

    @pl.when(sid < 8)
    def _():
        pltpu.sync_copy(sh_a.at[0, gslice], tmp_a)

        def cmb_a(j, _):
            pltpu.sync_copy(sh_a.at[j, gslice], tmp_b)
            for r in range(8):
                for c in range(8):
                    s_ = pl.ds(c * 16, 16)
                    tmp_a[r, s_] = jnp.maximum(tmp_a[r, s_], tmp_b[r, s_])
            return 0
        lax.fori_loop(1, _NS, cmb_a, 0)
        pltpu.sync_copy(tmp_a, gmax_glob.at[gslice])
    plsc.subcore_barrier()
    pltpu.sync_copy(gmax_glob, gmax_v)

    def row_b(jb, _):
        bvec = bbuf[pl.ds(8 * jb, 16)]
        for lane in range(8):
            g = bvec[lane]
            i = 8 * jb + lane
            for c in range(8):
                s_ = pl.ds(c * 16, 16)
                hv = hbuf[i, s_]
                e = jnp.exp(hv * tv - gmax_v[g, s_])
                s1_v[g, s_] = s1_v[g, s_] + e
                s2_v[g, s_] = s2_v[g, s_] + hv * e
        return 0
    for tile in range(2):
        ngt = stage(tile)
        lax.fori_loop(0, ngt, row_b, 0)

    pltpu.sync_copy(s1_v, sh_a.at[sid])
    pltpu.sync_copy(s2_v, sh_b.at[sid])
    plsc.subcore_barrier()

    @pl.when(sid < 8)
    def _():
        pltpu.sync_copy(sh_a.at[0, gslice], tmp_a)
        pltpu.sync_copy(sh_b.at[0, gslice], tmp_b)

        def cmb_b(j, _):
            pltpu.sync_copy(sh_a.at[j, gslice], s1_v.at[pl.ds(0, 8)])
            pltpu.sync_copy(sh_b.at[j, gslice], s2_v.at[pl.ds(0, 8)])
            for r in range(8):
                for c in range(8):
                    s_ = pl.ds(c * 16, 16)
                    tmp_a[r, s_] = tmp_a[r, s_] + s1_v[r, s_]
                    tmp_b[r, s_] = tmp_b[r, s_] + s2_v[r, s_]
            return 0
        lax.fori_loop(1, _NS, cmb_b, 0)
        for r in range(8):
            for c in range(8):
                s_ = pl.ds(c * 16, 16)
                tmp_b[r, s_] = tmp_b[r, s_] / (tmp_a[r, s_] + 1e-16)
        pltpu.sync_copy(tmp_b, out_hbm.at[gslice])


def _pool_body(h_hbm, b_hbm, t_hbm, out_hbm,
               hbuf, bbuf, gmax_v, s1_v, s2_v, tmp_a, tmp_b,
               tbuf, sh_a, sh_b, gmax_glob):
    cid = lax.axis_index("c")

    @pl.when(cid == 0)
    def _():
        _pool_core0(h_hbm, b_hbm, t_hbm, out_hbm,
                    hbuf, bbuf, gmax_v, s1_v, s2_v, tmp_a, tmp_b,
                    tbuf, sh_a, sh_b, gmax_glob)


@functools.cache
def _build_pool():
    return pl.kernel(
        _pool_body,
        out_type=jax.ShapeDtypeStruct((_NG, _H), jnp.float32),
        mesh=plsc.VectorSubcoreMesh(core_axis_name="c", subcore_axis_name="s",
                                    num_cores=_NC, num_subcores=_NS),
        scratch_types=[
            pltpu.VMEM((_TILE_R, _H), jnp.float32),
            pltpu.VMEM((_TILE_R + 16,), jnp.int32),
            pltpu.VMEM((_NG, _H), jnp.float32),
            pltpu.VMEM((_NG, _H), jnp.float32),
            pltpu.VMEM((_NG, _H), jnp.float32),
            pltpu.VMEM((8, _H), jnp.float32),
            pltpu.VMEM((8, _H), jnp.float32),
            pltpu.VMEM((16,), jnp.float32),
            pltpu.VMEM_SHARED((_NS, _NG, _H), jnp.float32),
            pltpu.VMEM_SHARED((_NS, _NG, _H), jnp.float32),
            pltpu.VMEM_SHARED((_NG, _H), jnp.float32),
        ],
    )


def _pool(h, batch, tvec):
    return _build_pool()(h, batch, tvec)


def kernel(x, edge_attr, global_feat, node_W, node_b, edge_W, edge_b,
           conv_lin_W, conv_lin_b, conv_W1, conv_b1, conv_W2, conv_b2, eps,
           glob_W1, glob_b1, glob_W2, glob_b2, aggr_t,
           head_W1, head_b1, head_W2, head_b2, head_W3, head_b3,
           edge_index, batch):
    f32 = jnp.float32
    src = edge_index[0]
    dst = edge_index[1]

    h = _node_embed(x, node_W.T, node_b.reshape(1, _H))

    ew_t = edge_W.T
    eb = edge_b.reshape(1, _H)
    for l in range(3):
        e_l = _edge_transform(edge_attr, ew_t, eb,
                              conv_lin_W[l].T, conv_lin_b[l].reshape(1, _H))
        aggr2 = _msgpass(h, e_l, src, dst)
        epsv = jnp.broadcast_to((1.0 + eps[l]).astype(f32).reshape(1, 1),
                                (1, _H))
        h = _node_update(h, aggr2, epsv,
                         conv_W1[l].T, conv_b1[l].reshape(1, _H),
                         conv_W2[l].T, conv_b2[l].reshape(1, _H))

    tvec = jnp.broadcast_to(aggr_t.astype(f32).reshape(1), (16,))
    hp = jnp.concatenate([h, jnp.zeros((16, _H), f32)], axis=0)
    bp = jnp.concatenate([batch, jnp.full((16,), _NG - 1, batch.dtype)])
    gemb = _pool(hp, bp, tvec)

    z = _head(gemb, global_feat,
              glob_W1.T, glob_b1.reshape(1, _H),
              glob_W2.T, glob_b2.reshape(1, _H),
              head_W1[:, :_H].T, head_W1[:, _H:].T, head_b1.reshape(1, 32),
              head_W2.T, head_b2.reshape(1, 16),
              head_W3.T, head_b3.reshape(1, 1))
    return z.reshape(-1)


# speedup vs baseline: 1.3566x; 1.3566x over previous
import functools

import jax
import jax.numpy as jnp
from jax import lax
from jax.experimental import pallas as pl
from jax.experimental.pallas import tpu as pltpu
from jax.experimental.pallas import tpu_sc as plsc

_N = 10000
_E = 320000
_H = 128
_NG = 64
_NC = 2
_NS = 16
_NW = _NC * _NS
_CK = 40
_NCH = _E // _CK
_CPW = _NCH // _NW
_NPW = 624
_NTAIL = _N - _NS * _NPW


def _node_embed_body(x_ref, w_ref, b_ref, o_ref):
    o_ref[...] = jnp.maximum(
        jnp.dot(x_ref[...], w_ref[...], preferred_element_type=jnp.float32)
        + b_ref[...], 0.0)


def _node_embed(x, w_t, b):
    return pl.pallas_call(
        _node_embed_body,
        grid=(10,),
        in_specs=[
            pl.BlockSpec((1000, _H), lambda i: (i, 0)),
            pl.BlockSpec((_H, _H), lambda i: (0, 0)),
            pl.BlockSpec((1, _H), lambda i: (0, 0)),
        ],
        out_specs=pl.BlockSpec((1000, _H), lambda i: (i, 0)),
        out_shape=jax.ShapeDtypeStruct((_N, _H), jnp.float32),
    )(x, w_t, b)


def _edge_body(ea_ref, w1_ref, b1_ref, w2_ref, b2_ref, o_ref):
    t = jnp.maximum(
        jnp.dot(ea_ref[...], w1_ref[...], preferred_element_type=jnp.float32)
        + b1_ref[...], 0.0)
    o_ref[...] = (
        jnp.dot(t, w2_ref[...], preferred_element_type=jnp.float32)
        + b2_ref[...])


def _edge_transform(ea, w1_t, b1, w2_t, b2):
    blk = 4000
    return pl.pallas_call(
        _edge_body,
        grid=(_E // blk,),
        in_specs=[
            pl.BlockSpec((blk, 16), lambda i: (i, 0)),
            pl.BlockSpec((16, _H), lambda i: (0, 0)),
            pl.BlockSpec((1, _H), lambda i: (0, 0)),
            pl.BlockSpec((_H, _H), lambda i: (0, 0)),
            pl.BlockSpec((1, _H), lambda i: (0, 0)),
        ],
        out_specs=pl.BlockSpec((blk, _H), lambda i: (i, 0)),
        out_shape=jax.ShapeDtypeStruct((_E, _H), jnp.float32),
    )(ea, w1_t, b1, w2_t, b2)


def _update_body(h_ref, a_ref, ev_ref, w1_ref, b1_ref, w2_ref, b2_ref, o_ref):
    u = h_ref[...] * ev_ref[...] + a_ref[0] + a_ref[1]
    t = jnp.maximum(
        jnp.dot(u, w1_ref[...], preferred_element_type=jnp.float32)
        + b1_ref[...], 0.0)
    o_ref[...] = jnp.maximum(
        jnp.dot(t, w2_ref[...], preferred_element_type=jnp.float32)
        + b2_ref[...], 0.0)


def _node_update(h, aggr2, epsv, w1_t, b1, w2_t, b2):
    return pl.pallas_call(
        _update_body,
        grid=(10,),
        in_specs=[
            pl.BlockSpec((1000, _H), lambda i: (i, 0)),
            pl.BlockSpec((_NC, 1000, _H), lambda i: (0, i, 0)),
            pl.BlockSpec((1, _H), lambda i: (0, 0)),
            pl.BlockSpec((_H, _H), lambda i: (0, 0)),
            pl.BlockSpec((1, _H), lambda i: (0, 0)),
            pl.BlockSpec((_H, _H), lambda i: (0, 0)),
            pl.BlockSpec((1, _H), lambda i: (0, 0)),
        ],
        out_specs=pl.BlockSpec((1000, _H), lambda i: (i, 0)),
        out_shape=jax.ShapeDtypeStruct((_N, _H), jnp.float32),
    )(h, aggr2, epsv, w1_t, b1, w2_t, b2)


def _head_body(ge_ref, gf_ref, gw1_ref, gb1_ref, gw2_ref, gb2_ref,
               w1a_ref, w1b_ref, b1_ref, w2_ref, b2_ref, w3_ref, b3_ref,
               o_ref):
    g = jnp.maximum(
        jnp.dot(gf_ref[...], gw1_ref[...], preferred_element_type=jnp.float32)
        + gb1_ref[...], 0.0)
    g = jnp.maximum(
        jnp.dot(g, gw2_ref[...], preferred_element_type=jnp.float32)
        + gb2_ref[...], 0.0)
    z = (jnp.dot(ge_ref[...], w1a_ref[...], preferred_element_type=jnp.float32)
         + jnp.dot(g, w1b_ref[...], preferred_element_type=jnp.float32)
         + b1_ref[...])
    z = jnp.maximum(z, 0.0)
    z = jnp.maximum(
        jnp.dot(z, w2_ref[...], preferred_element_type=jnp.float32)
        + b2_ref[...], 0.0)
    o_ref[...] = (
        jnp.dot(z, w3_ref[...], preferred_element_type=jnp.float32)
        + b3_ref[...])


def _head(ge, gf, gw1_t, gb1, gw2_t, gb2, w1a, w1b, b1, w2_t, b2, w3_t, b3):
    return pl.pallas_call(
        _head_body,
        out_shape=jax.ShapeDtypeStruct((_NG, 1), jnp.float32),
    )(ge, gf, gw1_t, gb1, gw2_t, gb2, w1a, w1b, b1, w2_t, b2, w3_t, b3)


def _msgpass_body(h_hbm, e_hbm, src1_hbm, dst1_hbm, out_hbm,
                  sg, sg2, dg, dg2, rows_v, msg_v, aggr_sh, sem_g, sem_e, sem_i):
    cid = lax.axis_index("c")
    sid = lax.axis_index("s")
    w = cid * _NS + sid

    zero16 = jnp.zeros((16,), jnp.float32)

    def zrow(i, _):
        for c in range(8):
            msg_v[0, i, pl.ds(c * 16, 16)] = zero16
        return 0
    lax.fori_loop(0, _CK, zrow, 0)
    base_n = sid * _NPW
    off_n = 0
    while off_n < _NPW:
        n = min(_CK, _NPW - off_n)
        pltpu.sync_copy(msg_v.at[0, pl.ds(0, n)],
                        aggr_sh.at[pl.ds(base_n + off_n, n)])
        off_n += n

    @pl.when(sid == 0)
    def _():
        pltpu.sync_copy(msg_v.at[0, pl.ds(0, _NTAIL)],
                        aggr_sh.at[pl.ds(_NS * _NPW, _NTAIL)])

    cb = _CPW * w
    sgs = (sg, sg2)
    dgs = (dg, dg2)

    def issue_idx(j, k):
        base = (cb + j) * _CK
        pltpu.async_copy(src1_hbm.at[pl.ds(base, _CK)], sgs[k], sem_i)
        pltpu.async_copy(dst1_hbm.at[pl.ds(base, _CK)], dgs[k], sem_i)

    def wait_idx(j, k):
        base = (cb + j) * _CK
        pltpu.make_async_copy(src1_hbm.at[pl.ds(base, _CK)], sgs[k],
                              sem_i).wait()
        pltpu.make_async_copy(dst1_hbm.at[pl.ds(base, _CK)], dgs[k],
                              sem_i).wait()

    def issue_in(j, k):
        pltpu.async_copy(h_hbm.at[sgs[k]], rows_v.at[k], sem_g)
        pltpu.async_copy(e_hbm.at[pl.ds((cb + j) * _CK, _CK)],
                         msg_v.at[k], sem_e)

    def drain_in(j, k):
        pltpu.make_async_copy(h_hbm.at[sgs[k]], rows_v.at[k], sem_g).wait()
        pltpu.make_async_copy(e_hbm.at[pl.ds((cb + j) * _CK, _CK)],
                              msg_v.at[k], sem_e).wait()

    def compute_scatter(k):
        def row(i, _):
            for c in range(8):
                s_ = pl.ds(c * 16, 16)
                msg_v[k, i, s_] = jnp.maximum(
                    rows_v[k, i, s_] + msg_v[k, i, s_], 0.0)
            return 0
        lax.fori_loop(0, _CK, row, 0)
        pltpu.sync_copy(msg_v.at[k], aggr_sh.at[dgs[k]], add=True)

    issue_idx(0, 0)
    wait_idx(0, 0)
    issue_in(0, 0)
    issue_idx(1, 1)
    plsc.subcore_barrier()

    def pair(jj, _):
        j = 2 * jj
        wait_idx(j + 1, 1)
        issue_in(j + 1, 1)
        drain_in(j, 0)

        @pl.when(j + 2 < _CPW)
        def _():
            issue_idx(j + 2, 0)
        compute_scatter(0)

        @pl.when(j + 2 < _CPW)
        def _():
            wait_idx(j + 2, 0)
            issue_in(j + 2, 0)
        drain_in(j + 1, 1)

        @pl.when(j + 3 < _CPW)
        def _():
            issue_idx(j + 3, 1)
        compute_scatter(1)
        return 0
    lax.fori_loop(0, _CPW // 2, pair, 0)

    plsc.subcore_barrier()
    pltpu.sync_copy(aggr_sh.at[pl.ds(base_n, _NPW)],
                    out_hbm.at[cid, pl.ds(base_n, _NPW)])

    @pl.when(sid == 0)
    def _():
        pltpu.sync_copy(aggr_sh.at[pl.ds(_NS * _NPW, _NTAIL)],
                        out_hbm.at[cid, pl.ds(_NS * _NPW, _NTAIL)])


@functools.cache
def _build_msgpass():
    return pl.kernel(
        _msgpass_body,
        out_type=jax.ShapeDtypeStruct((_NC, _N, _H), jnp.float32),
        mesh=plsc.VectorSubcoreMesh(core_axis_name="c", subcore_axis_name="s",
                                    num_cores=_NC, num_subcores=_NS),
        scratch_types=[
            pltpu.VMEM((_CK,), jnp.int32),
            pltpu.VMEM((_CK,), jnp.int32),
            pltpu.VMEM((_CK,), jnp.int32),
            pltpu.VMEM((_CK,), jnp.int32),
            pltpu.VMEM((2, _CK, _H), jnp.float32),
            pltpu.VMEM((2, _CK, _H), jnp.float32),
            pltpu.VMEM_SHARED((_N, _H), jnp.float32),
            pltpu.SemaphoreType.DMA,
            pltpu.SemaphoreType.DMA,
            pltpu.SemaphoreType.DMA,
        ],
    )


def _msgpass(h, e_l, src1, dst1):
    return _build_msgpass()(h, e_l, src1, dst1)


_TILE_G = 40
_TILE_R = _TILE_G * 8


def _pool_core0(h_hbm, b_hbm, t_hbm, out_hbm,
                hbuf, bbuf, gmax_v, s1_v, s2_v, tmp_a, tmp_b,
                tbuf, sh_a, sh_b, gmax_glob):
    sid = lax.axis_index("s")
    gcnt = 78 + jnp.where(sid < 2, 1, 0)
    gbase = 78 * sid + jnp.minimum(sid, 2)
    rbase = gbase * 8

    pltpu.sync_copy(t_hbm, tbuf)
    tv = tbuf[...]

    zero16 = jnp.zeros((16,), jnp.float32)
    neg16 = jnp.full((16,), -3e38, jnp.float32)

    def zi(i, _):
        for c in range(8):
            s_ = pl.ds(c * 16, 16)
            gmax_v[i, s_] = neg16
            s1_v[i, s_] = zero16
            s2_v[i, s_] = zero16
        return 0
    lax.fori_loop(0, _NG, zi, 0)

    def stage(tile):
        rs = rbase + _TILE_R * tile
        pltpu.sync_copy(h_hbm.at[pl.ds(rs, _TILE_R)], hbuf)
        pltpu.sync_copy(b_hbm.at[pl.ds(rs, _TILE_R)],
                        bbuf.at[pl.ds(0, _TILE_R)])
        return jnp.minimum(_TILE_G, gcnt - _TILE_G * tile)

    def row_a(jb, _):
        bvec = bbuf[pl.ds(8 * jb, 16)]
        for lane in range(8):
            g = bvec[lane]
            i = 8 * jb + lane
            for c in range(8):
                s_ = pl.ds(c * 16, 16)
                gmax_v[g, s_] = jnp.maximum(gmax_v[g, s_], hbuf[i, s_] * tv)
        return 0
    for tile in range(2):
        ngt = stage(tile)
        lax.fori_loop(0, ngt, row_a, 0)

    pltpu.sync_copy(gmax_v, sh_a.at[sid])
    plsc.subcore_barrier()

    gslice = pl.ds(8 * sid, 8)

    @pl.when(sid < 8)
    def _():
        pltpu.sync_copy(sh_a.at[0, gslice], tmp_a)

        def cmb_a(j, _):
            pltpu.sync_copy(sh_a.at[j, gslice], tmp_b)
            for r in range(8):
                for c in range(8):
                    s_ = pl.ds(c * 16, 16)
                    tmp_a[r, s_] = jnp.maximum(tmp_a[r, s_], tmp_b[r, s_])
            return 0
        lax.fori_loop(1, _NS, cmb_a, 0)
        pltpu.sync_copy(tmp_a, gmax_glob.at[gslice])
    plsc.subcore_barrier()
    pltpu.sync_copy(gmax_glob, gmax_v)

    def row_b(jb, _):
        bvec = bbuf[pl.ds(8 * jb, 16)]
        for lane in range(8):
            g = bvec[lane]
            i = 8 * jb + lane
            for c in range(8):
                s_ = pl.ds(c * 16, 16)
                hv = hbuf[i, s_]
                e = jnp.exp(hv * tv - gmax_v[g, s_])
                s1_v[g, s_] = s1_v[g, s_] + e
                s2_v[g, s_] = s2_v[g, s_] + hv * e
        return 0
    for tile in range(2):
        ngt = stage(tile)
        lax.fori_loop(0, ngt, row_b, 0)

    pltpu.sync_copy(s1_v, sh_a.at[sid])
    pltpu.sync_copy(s2_v, sh_b.at[sid])
    plsc.subcore_barrier()

    @pl.when(sid < 8)
    def _():
        pltpu.sync_copy(sh_a.at[0, gslice], tmp_a)
        pltpu.sync_copy(sh_b.at[0, gslice], tmp_b)

        def cmb_b(j, _):
            pltpu.sync_copy(sh_a.at[j, gslice], s1_v.at[pl.ds(0, 8)])
            pltpu.sync_copy(sh_b.at[j, gslice], s2_v.at[pl.ds(0, 8)])
            for r in range(8):
                for c in range(8):
                    s_ = pl.ds(c * 16, 16)
                    tmp_a[r, s_] = tmp_a[r, s_] + s1_v[r, s_]
                    tmp_b[r, s_] = tmp_b[r, s_] + s2_v[r, s_]
            return 0
        lax.fori_loop(1, _NS, cmb_b, 0)
        for r in range(8):
            for c in range(8):
                s_ = pl.ds(c * 16, 16)
                tmp_b[r, s_] = tmp_b[r, s_] / (tmp_a[r, s_] + 1e-16)
        pltpu.sync_copy(tmp_b, out_hbm.at[gslice])


def _pool_body(h_hbm, b_hbm, t_hbm, out_hbm,
               hbuf, bbuf, gmax_v, s1_v, s2_v, tmp_a, tmp_b,
               tbuf, sh_a, sh_b, gmax_glob):
    cid = lax.axis_index("c")

    @pl.when(cid == 0)
    def _():
        _pool_core0(h_hbm, b_hbm, t_hbm, out_hbm,
                    hbuf, bbuf, gmax_v, s1_v, s2_v, tmp_a, tmp_b,
                    tbuf, sh_a, sh_b, gmax_glob)


@functools.cache
def _build_pool():
    return pl.kernel(
        _pool_body,
        out_type=jax.ShapeDtypeStruct((_NG, _H), jnp.float32),
        mesh=plsc.VectorSubcoreMesh(core_axis_name="c", subcore_axis_name="s",
                                    num_cores=_NC, num_subcores=_NS),
        scratch_types=[
            pltpu.VMEM((_TILE_R, _H), jnp.float32),
            pltpu.VMEM((_TILE_R + 16,), jnp.int32),
            pltpu.VMEM((_NG, _H), jnp.float32),
            pltpu.VMEM((_NG, _H), jnp.float32),
            pltpu.VMEM((_NG, _H), jnp.float32),
            pltpu.VMEM((8, _H), jnp.float32),
            pltpu.VMEM((8, _H), jnp.float32),
            pltpu.VMEM((16,), jnp.float32),
            pltpu.VMEM_SHARED((_NS, _NG, _H), jnp.float32),
            pltpu.VMEM_SHARED((_NS, _NG, _H), jnp.float32),
            pltpu.VMEM_SHARED((_NG, _H), jnp.float32),
        ],
    )


def _pool(h, batch, tvec):
    return _build_pool()(h, batch, tvec)


def kernel(x, edge_attr, global_feat, node_W, node_b, edge_W, edge_b,
           conv_lin_W, conv_lin_b, conv_W1, conv_b1, conv_W2, conv_b2, eps,
           glob_W1, glob_b1, glob_W2, glob_b2, aggr_t,
           head_W1, head_b1, head_W2, head_b2, head_W3, head_b3,
           edge_index, batch):
    f32 = jnp.float32
    src = edge_index[0]
    dst = edge_index[1]

    h = _node_embed(x, node_W.T, node_b.reshape(1, _H))

    ew_t = edge_W.T
    eb = edge_b.reshape(1, _H)
    for l in range(3):
        e_l = _edge_transform(edge_attr, ew_t, eb,
                              conv_lin_W[l].T, conv_lin_b[l].reshape(1, _H))
        aggr2 = _msgpass(h, e_l, src, dst)
        epsv = jnp.broadcast_to((1.0 + eps[l]).astype(f32).reshape(1, 1),
                                (1, _H))
        h = _node_update(h, aggr2, epsv,
                         conv_W1[l].T, conv_b1[l].reshape(1, _H),
                         conv_W2[l].T, conv_b2[l].reshape(1, _H))

    tvec = jnp.broadcast_to(aggr_t.astype(f32).reshape(1), (16,))
    hp = jnp.concatenate([h, jnp.zeros((16, _H), f32)], axis=0)
    bp = jnp.concatenate([batch, jnp.full((16,), _NG - 1, batch.dtype)])
    gemb = _pool(hp, bp, tvec)

    z = _head(gemb, global_feat,
              glob_W1.T, glob_b1.reshape(1, _H),
              glob_W2.T, glob_b2.reshape(1, _H),
              head_W1[:, :_H].T, head_W1[:, _H:].T, head_b1.reshape(1, 32),
              head_W2.T, head_b2.reshape(1, 16),
              head_W3.T, head_b3.reshape(1, 1))
    return z.reshape(-1)
